# baseline (speedup 1.0000x reference)
import functools

import jax
import jax.numpy as jnp
from jax import lax
from jax.experimental import pallas as pl
from jax.experimental.pallas import tpu as pltpu
from jax.experimental.pallas import tpu_sc as plsc

_NC = 2
_NS = 16
_NW = _NC * _NS
_LANES = 16
_R = 32


def _sc_add(inputs, pos):
    B, S, D = inputs.shape
    s_per_w = S // _NW
    n_chunks = s_per_w // _R
    mesh = plsc.VectorSubcoreMesh(core_axis_name="c", subcore_axis_name="s")

    def body(in_hbm, emb_hbm, out_hbm, ebuf, buf):
        wid = lax.axis_index("s") * _NC + lax.axis_index("c")
        base = wid * s_per_w

        def chunk_loop(i, carry):
            s0 = base + i * _R
            pltpu.sync_copy(emb_hbm.at[pl.ds(s0, _R)], ebuf)

            def batch_loop(b, carry2):
                pltpu.sync_copy(in_hbm.at[b, pl.ds(s0, _R)], buf)

                def row_loop(r, carry3):
                    for c in range(D // _LANES):
                        sl = pl.ds(c * _LANES, _LANES)
                        plsc.addupdate(buf.at[r, sl], ebuf[r, sl])
                    return carry3

                lax.fori_loop(0, _R, row_loop, 0)
                pltpu.sync_copy(buf, out_hbm.at[b, pl.ds(s0, _R)])
                return carry2

            lax.fori_loop(0, B, batch_loop, 0)
            return carry

        lax.fori_loop(0, n_chunks, chunk_loop, 0)

    f = pl.kernel(
        body,
        out_type=jax.ShapeDtypeStruct((B, S, D), jnp.float32),
        mesh=mesh,
        scratch_types=[
            pltpu.VMEM((_R, D), jnp.float32),
            pltpu.VMEM((_R, D), jnp.float32),
        ],
    )
    return f(inputs, pos)


def kernel(inputs, embeddings):
    B, S, D = inputs.shape
    pos = embeddings[:S]
    return _sc_add(inputs, pos)


# speedup vs baseline: 3.8314x; 3.8314x over previous
import jax
import jax.numpy as jnp
from jax.experimental import pallas as pl
from jax.experimental.pallas import tpu as pltpu

_RB = 1024


def _add_body(in_ref, emb_ref, out_ref):
    out_ref[...] = in_ref[...] + emb_ref[...]


def kernel(inputs, embeddings):
    B, S, D = inputs.shape
    pos = embeddings[:S]
    flat = inputs.reshape(B * S, D)
    n_s = S // _RB
    out = pl.pallas_call(
        _add_body,
        grid=(n_s, B),
        in_specs=[
            pl.BlockSpec((_RB, D), lambda s, b: (b * n_s + s, 0)),
            pl.BlockSpec((_RB, D), lambda s, b: (s, 0)),
        ],
        out_specs=pl.BlockSpec((_RB, D), lambda s, b: (b * n_s + s, 0)),
        out_shape=jax.ShapeDtypeStruct((B * S, D), inputs.dtype),
        compiler_params=pltpu.CompilerParams(
            dimension_semantics=("arbitrary", "arbitrary"),
        ),
    )(flat, pos)
    return out.reshape(B, S, D)


# speedup vs baseline: 3.9711x; 1.0365x over previous
import jax
import jax.numpy as jnp
from jax.experimental import pallas as pl
from jax.experimental.pallas import tpu as pltpu

_BS = 512


def _add_body(in_ref, emb_ref, out_ref):
    out_ref[...] = in_ref[...] + emb_ref[...][None]


def kernel(inputs, embeddings):
    B, S, D = inputs.shape
    pos = embeddings[:S]
    n_s = S // _BS
    return pl.pallas_call(
        _add_body,
        grid=(n_s,),
        in_specs=[
            pl.BlockSpec((B, _BS, D), lambda s: (0, s, 0)),
            pl.BlockSpec((_BS, D), lambda s: (s, 0)),
        ],
        out_specs=pl.BlockSpec((B, _BS, D), lambda s: (0, s, 0)),
        out_shape=jax.ShapeDtypeStruct((B, S, D), inputs.dtype),
        compiler_params=pltpu.CompilerParams(
            dimension_semantics=("arbitrary",),
        ),
    )(inputs, pos)


# speedup vs baseline: 3.9732x; 1.0005x over previous
import jax
import jax.numpy as jnp
from jax.experimental import pallas as pl
from jax.experimental.pallas import tpu as pltpu

_BS = 512


def _add_body(in_ref, emb_ref, out_ref):
    out_ref[...] = in_ref[...] + emb_ref[...][None]


def kernel(inputs, embeddings):
    B, S, D = inputs.shape
    pos = embeddings[:S]
    n_s = S // 1024
    return pl.pallas_call(
        _add_body,
        grid=(n_s, 2),
        in_specs=[
            pl.BlockSpec((2, 1024, D), lambda s, b: (b, s, 0)),
            pl.BlockSpec((1024, D), lambda s, b: (s, 0)),
        ],
        out_specs=pl.BlockSpec((2, 1024, D), lambda s, b: (b, s, 0)),
        out_shape=jax.ShapeDtypeStruct((B, S, D), inputs.dtype),
        compiler_params=pltpu.CompilerParams(
            dimension_semantics=("arbitrary", "arbitrary"),
        ),
    )(inputs, pos)
